# Initial kernel scaffold; baseline (speedup 1.0000x reference)
import jax
import jax.numpy as jnp
from jax.experimental import pallas as pl


def kernel(batch, pred, true):
    raise NotImplementedError("write your pallas kernel here")


# speedup vs baseline: 13.4003x; 13.4003x over previous
import jax
import jax.numpy as jnp
from jax.experimental import pallas as pl
from jax.experimental.pallas import tpu as pltpu

_N = 8388608
_W = 672
_COLS = 1024
_ROWS = _N // _COLS
_BLK_R = 256
_GRID = _ROWS // _BLK_R
_SHIFT_C = _COLS - _W


def _body(b_ref, p_ref, t_ref, out_ref, carry_ref):
    k = pl.program_id(0)

    @pl.when(k == 0)
    def _init():
        carry_ref[...] = jnp.full((1, _COLS), -1, jnp.int32)
        out_ref[...] = jnp.zeros((1, 1), jnp.float32)

    b = b_ref[...]
    aug = jnp.concatenate([carry_ref[...], b], axis=0)
    shifted = jnp.concatenate(
        [aug[:_BLK_R, _SHIFT_C:], aug[1:, :_SHIFT_C]], axis=1)
    w = jnp.where(b != shifted, 10.0, 1.0).astype(jnp.float32)
    d = p_ref[...] - t_ref[...]
    s = jnp.sum(w * (d * d)).reshape(1, 1)
    out_ref[...] = out_ref[...] + s
    carry_ref[...] = b[_BLK_R - 1:_BLK_R, :]


def kernel(batch, pred, true):
    b2 = batch.astype(jnp.int32).reshape(_ROWS, _COLS)
    p2 = pred.reshape(_ROWS, _COLS)
    t2 = true.reshape(_ROWS, _COLS)
    spec = pl.BlockSpec((_BLK_R, _COLS), lambda k: (k, 0))
    total = pl.pallas_call(
        _body,
        grid=(_GRID,),
        in_specs=[spec, spec, spec],
        out_specs=pl.BlockSpec((1, 1), lambda k: (0, 0)),
        out_shape=jax.ShapeDtypeStruct((1, 1), jnp.float32),
        scratch_shapes=[pltpu.VMEM((1, _COLS), jnp.int32)],
    )(b2, p2, t2)
    return total[0, 0] / _N


# speedup vs baseline: 42.0488x; 3.1379x over previous
import jax
import jax.numpy as jnp
from jax.experimental import pallas as pl
from jax.experimental.pallas import tpu as pltpu

_N = 8388608
_LANES = 128
_ROWS = _N // _LANES
_BLK_R = 2048
_GRID = _ROWS // _BLK_R
_CARRY_R = 6


def _body(b_ref, p_ref, t_ref, out_ref, carry_ref):
    k = pl.program_id(0)

    @pl.when(k == 0)
    def _init():
        carry_ref[...] = jnp.full((_CARRY_R, _LANES), -1, jnp.int32)
        out_ref[...] = jnp.zeros((1, 1), jnp.float32)

    b = b_ref[...]
    aug = jnp.concatenate([carry_ref[...], b], axis=0)
    shifted = jnp.concatenate(
        [aug[:_BLK_R, 96:], aug[1:_BLK_R + 1, :96]], axis=1)
    w = jnp.where(b != shifted, 10.0, 1.0).astype(jnp.float32)
    d = p_ref[...] - t_ref[...]
    s = jnp.sum(w * (d * d)).reshape(1, 1)
    out_ref[...] = out_ref[...] + s
    carry_ref[...] = b[_BLK_R - _CARRY_R:_BLK_R, :]


def kernel(batch, pred, true):
    b2 = batch.astype(jnp.int32).reshape(_ROWS, _LANES)
    p2 = pred.reshape(_ROWS, _LANES)
    t2 = true.reshape(_ROWS, _LANES)
    spec = pl.BlockSpec((_BLK_R, _LANES), lambda k: (k, 0))
    total = pl.pallas_call(
        _body,
        grid=(_GRID,),
        in_specs=[spec, spec, spec],
        out_specs=pl.BlockSpec((1, 1), lambda k: (0, 0)),
        out_shape=jax.ShapeDtypeStruct((1, 1), jnp.float32),
        scratch_shapes=[pltpu.VMEM((_CARRY_R, _LANES), jnp.int32)],
    )(b2, p2, t2)
    return total[0, 0] / _N


# speedup vs baseline: 51.4665x; 1.2240x over previous
import jax
import jax.numpy as jnp
from jax.experimental import pallas as pl
from jax.experimental.pallas import tpu as pltpu

_N = 8388608
_LANES = 128
_ROWS = _N // _LANES
_BLK_R = 4096
_GRID = _ROWS // _BLK_R
_CARRY_R = 6


def _body(b_ref, p_ref, t_ref, out_ref, carry_ref):
    k = pl.program_id(0)

    @pl.when(k == 0)
    def _init():
        carry_ref[...] = jnp.full((_CARRY_R, _LANES), -1, jnp.int32)
        out_ref[...] = jnp.zeros((1, 1), jnp.float32)

    b = b_ref[...]
    aug = jnp.concatenate([carry_ref[...], b], axis=0)
    shifted = jnp.concatenate(
        [aug[:_BLK_R, 96:], aug[1:_BLK_R + 1, :96]], axis=1)
    w = jnp.where(b != shifted, 10.0, 1.0).astype(jnp.float32)
    d = p_ref[...] - t_ref[...]
    s = jnp.sum(w * (d * d)).reshape(1, 1)
    out_ref[...] = out_ref[...] + s
    carry_ref[...] = b[_BLK_R - _CARRY_R:_BLK_R, :]


def kernel(batch, pred, true):
    b2 = batch.astype(jnp.int32).reshape(_ROWS, _LANES)
    p2 = pred.reshape(_ROWS, _LANES)
    t2 = true.reshape(_ROWS, _LANES)
    spec = pl.BlockSpec((_BLK_R, _LANES), lambda k: (k, 0))
    total = pl.pallas_call(
        _body,
        grid=(_GRID,),
        in_specs=[spec, spec, spec],
        out_specs=pl.BlockSpec((1, 1), lambda k: (0, 0)),
        out_shape=jax.ShapeDtypeStruct((1, 1), jnp.float32),
        scratch_shapes=[pltpu.VMEM((_CARRY_R, _LANES), jnp.int32)],
    )(b2, p2, t2)
    return total[0, 0] / _N


# speedup vs baseline: 56.1238x; 1.0905x over previous
import jax
import jax.numpy as jnp
from jax.experimental import pallas as pl
from jax.experimental.pallas import tpu as pltpu

_N = 8388608
_LANES = 128
_ROWS = _N // _LANES
_BLK_R = 8192
_GRID = _ROWS // _BLK_R
_CARRY_R = 6


def _body(b_ref, p_ref, t_ref, out_ref, carry_ref):
    k = pl.program_id(0)

    @pl.when(k == 0)
    def _init():
        carry_ref[...] = jnp.full((_CARRY_R, _LANES), -1, jnp.int32)
        out_ref[...] = jnp.zeros((1, 1), jnp.float32)

    b = b_ref[...]
    aug = jnp.concatenate([carry_ref[...], b], axis=0)
    shifted = jnp.concatenate(
        [aug[:_BLK_R, 96:], aug[1:_BLK_R + 1, :96]], axis=1)
    w = jnp.where(b != shifted, 10.0, 1.0).astype(jnp.float32)
    d = p_ref[...] - t_ref[...]
    s = jnp.sum(w * (d * d)).reshape(1, 1)
    out_ref[...] = out_ref[...] + s
    carry_ref[...] = b[_BLK_R - _CARRY_R:_BLK_R, :]


def kernel(batch, pred, true):
    b2 = batch.astype(jnp.int32).reshape(_ROWS, _LANES)
    p2 = pred.reshape(_ROWS, _LANES)
    t2 = true.reshape(_ROWS, _LANES)
    spec = pl.BlockSpec((_BLK_R, _LANES), lambda k: (k, 0))
    total = pl.pallas_call(
        _body,
        grid=(_GRID,),
        in_specs=[spec, spec, spec],
        out_specs=pl.BlockSpec((1, 1), lambda k: (0, 0)),
        out_shape=jax.ShapeDtypeStruct((1, 1), jnp.float32),
        scratch_shapes=[pltpu.VMEM((_CARRY_R, _LANES), jnp.int32)],
    )(b2, p2, t2)
    return total[0, 0] / _N
